# baseline (speedup 1.0000x reference)
import functools

import jax
import jax.numpy as jnp
from jax.experimental import pallas as pl
from jax.experimental.pallas import tpu as pltpu

_N = 10000
_E = 2000
_D = 128
_CHN = 400
_CHE = 80
_NSTEP = 25
_DN = (((0,), (0,)), ((), ()))
_DNAT = (((1,), (0,)), ((), ()))
_F32 = jnp.float32


def _hypersage_kernel(x_ref, inct_ref, w1_ref, w2_ref, wlin_ref, blin_ref,
                      out_ref, inct_bf, x2s, t_acc, ehi, ide, pooled):
    p = pl.program_id(0)
    i = pl.program_id(1)
    bf16 = jnp.bfloat16
    rn = pl.ds(i * _CHN, _CHN)
    re = pl.ds(i * _CHE, _CHE)

    @pl.when(p == 0)
    def _():
        f = x_ref[...]
        x2s[rn, :] = (f * f).astype(bf16)

    @pl.when(p == 1)
    def _():
        blk = inct_ref[...]
        bb = blk.astype(bf16)
        inct_bf[re, :] = bb
        inv_deg = 1.0 / jnp.sum(blk, axis=1, keepdims=True)
        ide[re, :] = inv_deg
        s = jax.lax.dot_general(bb, x2s[...], _DNAT,
                                preferred_element_type=_F32)
        ehi[re, :] = (s * inv_deg).astype(bf16)

    def accum_t(first):
        acc = jax.lax.dot_general(inct_bf[re, :], ehi[re, :], _DN,
                                  preferred_element_type=_F32)

        @pl.when(first)
        def _():
            t_acc[...] = acc

        @pl.when(jnp.logical_not(first))
        def _():
            t_acc[...] += acc

    @pl.when(p == 2)
    def _():
        accum_t(i == 0)

    def finalize(W):
        h = jax.lax.dot_general(jnp.sqrt(t_acc[rn, :]), W, _DNAT,
                                preferred_element_type=_F32)
        h = jnp.maximum(h, 0.0)
        norm = jnp.sqrt(jnp.sum(h * h, axis=-1, keepdims=True))
        return h / (norm + 1e-12)

    @pl.when(p == 3)
    def _():
        h = finalize(w1_ref[...])
        x2s[rn, :] = (h * h).astype(bf16)

    @pl.when(p == 4)
    def _():
        s = jax.lax.dot_general(inct_bf[re, :], x2s[...], _DNAT,
                                preferred_element_type=_F32)
        ehi[re, :] = (s * ide[re, :]).astype(bf16)

    @pl.when(p == 5)
    def _():
        accum_t(i == 0)

    @pl.when(jnp.logical_and(p == 6, i == 0))
    def _():
        pooled[...] = jnp.full((1, _D), -jnp.inf, _F32)

    @pl.when(p == 6)
    def _():
        h = finalize(w2_ref[...])
        pooled[...] = jnp.maximum(pooled[...],
                                  jnp.max(h, axis=0, keepdims=True))

    @pl.when(jnp.logical_and(p == 6, i == _NSTEP - 1))
    def _():
        dn_t = (((1,), (1,)), ((), ()))
        out_ref[...] = (
            jax.lax.dot_general(pooled[...], wlin_ref[...], dn_t,
                                preferred_element_type=_F32)
            + blin_ref[...])


@jax.jit
def kernel(x_0, incidence, W1, W2, Wlin, b_lin):
    grid = (7, _NSTEP)

    def x_map(p, i):
        return (jnp.where(p == 0, i, 0), 0)

    def inct_map(p, i):
        return (jnp.where(p == 1, i, 0), 0)

    def const_map(p, i):
        return (0, 0)

    out = pl.pallas_call(
        _hypersage_kernel,
        grid=grid,
        in_specs=[
            pl.BlockSpec((_CHN, _D), x_map),
            pl.BlockSpec((_CHE, _N), inct_map),
            pl.BlockSpec((_D, _D), const_map),
            pl.BlockSpec((_D, _D), const_map),
            pl.BlockSpec((_D, _D), const_map),
            pl.BlockSpec((1, _D), const_map),
        ],
        out_specs=pl.BlockSpec((1, _D), const_map),
        out_shape=jax.ShapeDtypeStruct((1, _D), jnp.float32),
        scratch_shapes=[
            pltpu.VMEM((_E, _N), jnp.bfloat16),
            pltpu.VMEM((_N, _D), jnp.bfloat16),
            pltpu.VMEM((_N, _D), _F32),
            pltpu.VMEM((_E, _D), jnp.bfloat16),
            pltpu.VMEM((_E, 1), _F32),
            pltpu.VMEM((1, _D), _F32),
        ],
        compiler_params=pltpu.CompilerParams(
            vmem_limit_bytes=64 * 1024 * 1024,
        ),
    )(x_0, incidence.T, W1, W2, Wlin, b_lin.reshape(1, -1))
    return out.reshape(-1)


# speedup vs baseline: 1.9807x; 1.9807x over previous
import functools

import jax
import jax.numpy as jnp
from jax.experimental import pallas as pl
from jax.experimental.pallas import tpu as pltpu

_N = 10000
_NP = 10240
_E = 2000
_D = 128
_CHN = 400
_CHE = 80
_CHL = 1280
_NSTEP = 25
_NL = _NP // _CHL
_DNAT = (((1,), (0,)), ((), ()))
_F32 = jnp.float32


def _hypersage_kernel(x_ref, inct_ref, w1t_ref, w2t_ref, wlin_ref, blin_ref,
                      out_ref, inct_bf, x2s, ehi, ehit, ide, pooled):
    p = pl.program_id(0)
    i = pl.program_id(1)
    bf16 = jnp.bfloat16
    rn = pl.ds(i * _CHN, _CHN)
    re = pl.ds(i * _CHE, _CHE)

    @pl.when(jnp.logical_and(p == 0, i == 0))
    def _():
        x2s[pl.ds(_N, _NP - _N), :] = jnp.zeros((_NP - _N, _D), bf16)

    @pl.when(p == 0)
    def _():
        f = x_ref[...]
        x2s[rn, :] = (f * f).astype(bf16)

    @pl.when(p == 1)
    def _():
        blk = inct_ref[...]
        bbp = jnp.concatenate(
            [blk.astype(bf16), jnp.zeros((_CHE, _NP - _N), bf16)], axis=1)
        inct_bf[re, :] = bbp
        inv_deg = 1.0 / jnp.sum(blk, axis=1, keepdims=True)
        ide[re, :] = inv_deg
        s = jax.lax.dot_general(bbp, x2s[...], _DNAT,
                                preferred_element_type=_F32)
        ehi[re, :] = (s * inv_deg).astype(bf16)

    def inter_edge(Wt):
        rl = pl.ds(i * _CHL, _CHL)
        tt = jax.lax.dot_general(ehit[...], inct_bf[:, rl], _DNAT,
                                 preferred_element_type=_F32)
        ht = jax.lax.dot_general(Wt, jnp.sqrt(tt), _DNAT,
                                 preferred_element_type=_F32)
        ht = jnp.maximum(ht, 0.0)
        norm = jnp.sqrt(jnp.sum(ht * ht, axis=0, keepdims=True))
        return ht / (norm + 1e-12)

    @pl.when(jnp.logical_and(p == 2, i == 0))
    def _():
        ehit[...] = jnp.transpose(ehi[...])

    @pl.when(jnp.logical_and(p == 2, i < _NL))
    def _():
        ht = inter_edge(w1t_ref[...])
        x2s[pl.ds(i * _CHL, _CHL), :] = jnp.transpose(
            (ht * ht).astype(bf16))

    @pl.when(p == 3)
    def _():
        s = jax.lax.dot_general(inct_bf[re, :], x2s[...], _DNAT,
                                preferred_element_type=_F32)
        ehi[re, :] = (s * ide[re, :]).astype(bf16)

    @pl.when(jnp.logical_and(p == 4, i == 0))
    def _():
        ehit[...] = jnp.transpose(ehi[...])
        pooled[...] = jnp.zeros((_D, 1), _F32)

    @pl.when(jnp.logical_and(p == 4, i < _NL))
    def _():
        ht = inter_edge(w2t_ref[...])
        pooled[...] = jnp.maximum(pooled[...],
                                  jnp.max(ht, axis=1, keepdims=True))

    @pl.when(jnp.logical_and(p == 4, i == _NSTEP - 1))
    def _():
        out_ref[...] = (
            jax.lax.dot_general(wlin_ref[...], pooled[...], _DNAT,
                                preferred_element_type=_F32)
            + blin_ref[...])


@jax.jit
def kernel(x_0, incidence, W1, W2, Wlin, b_lin):
    grid = (5, _NSTEP)

    def x_map(p, i):
        return (jnp.where(p == 0, i, 0), 0)

    def inct_map(p, i):
        return (jnp.where(p == 1, i, 0), 0)

    def const_map(p, i):
        return (0, 0)

    out = pl.pallas_call(
        _hypersage_kernel,
        grid=grid,
        in_specs=[
            pl.BlockSpec((_CHN, _D), x_map),
            pl.BlockSpec((_CHE, _N), inct_map),
            pl.BlockSpec((_D, _D), const_map),
            pl.BlockSpec((_D, _D), const_map),
            pl.BlockSpec((_D, _D), const_map),
            pl.BlockSpec((_D, 1), const_map),
        ],
        out_specs=pl.BlockSpec((_D, 1), const_map),
        out_shape=jax.ShapeDtypeStruct((_D, 1), jnp.float32),
        scratch_shapes=[
            pltpu.VMEM((_E, _NP), jnp.bfloat16),
            pltpu.VMEM((_NP, _D), jnp.bfloat16),
            pltpu.VMEM((_E, _D), jnp.bfloat16),
            pltpu.VMEM((_D, _E), jnp.bfloat16),
            pltpu.VMEM((_E, 1), _F32),
            pltpu.VMEM((_D, 1), _F32),
        ],
        compiler_params=pltpu.CompilerParams(
            vmem_limit_bytes=64 * 1024 * 1024,
        ),
    )(x_0, incidence.T, W1.T, W2.T, Wlin, b_lin.reshape(-1, 1))
    return out.reshape(-1)


# speedup vs baseline: 2.3682x; 1.1956x over previous
import functools

import jax
import jax.numpy as jnp
from jax.experimental import pallas as pl
from jax.experimental.pallas import tpu as pltpu

_N = 10000
_NP = 10240
_E = 2000
_D = 128
_CHN = 1000
_CHE = 80
_CHL = 1280
_NL = _NP // _CHL
_NX = _N // _CHN
_NE = _E // _CHE
_S_STREAM = _NX
_S_T1 = _S_STREAM + _NE
_S_L1 = _S_T1 + 1
_S_S2 = _S_L1 + _NL
_S_T2 = _S_S2 + 1
_S_L2 = _S_T2 + 1
_S_HEAD = _S_L2 + _NL
_STEPS = _S_HEAD + 1
_DNAT = (((1,), (0,)), ((), ()))
_F32 = jnp.float32


def _hypersage_kernel(x_ref, inct_ref, w1t_ref, w2t_ref, wlin_ref, blin_ref,
                      out_ref, inct_bf, x2s, ehi, ehit, ide, pooled):
    i = pl.program_id(0)
    bf16 = jnp.bfloat16

    @pl.when(i == 0)
    def _():
        x2s[pl.ds(_N, _NP - _N), :] = jnp.zeros((_NP - _N, _D), bf16)

    @pl.when(i < _S_STREAM)
    def _():
        f = x_ref[...]
        x2s[pl.ds(i * _CHN, _CHN), :] = (f * f).astype(bf16)

    @pl.when(jnp.logical_and(i >= _S_STREAM, i < _S_T1))
    def _():
        e = i - _S_STREAM
        re = pl.ds(e * _CHE, _CHE)
        blk = inct_ref[...]
        bbp = jnp.concatenate(
            [blk.astype(bf16), jnp.zeros((_CHE, _NP - _N), bf16)], axis=1)
        inct_bf[re, :] = bbp
        inv_deg = 1.0 / jnp.sum(blk, axis=1, keepdims=True)
        ide[re, :] = inv_deg
        s = jax.lax.dot_general(bbp, x2s[...], _DNAT,
                                preferred_element_type=_F32)
        ehi[re, :] = (s * inv_deg).astype(bf16)

    @pl.when(jnp.logical_or(i == _S_T1, i == _S_T2))
    def _():
        ehit[...] = jnp.transpose(ehi[...])

    def inter_edge(Wt, j):
        rl = pl.ds(j * _CHL, _CHL)
        tt = jax.lax.dot_general(ehit[...], inct_bf[:, rl], _DNAT,
                                 preferred_element_type=_F32)
        ht = jax.lax.dot_general(Wt, jnp.sqrt(tt), _DNAT,
                                 preferred_element_type=_F32)
        ht = jnp.maximum(ht, 0.0)
        norm = jnp.sqrt(jnp.sum(ht * ht, axis=0, keepdims=True))
        return ht / (norm + 1e-12)

    @pl.when(jnp.logical_and(i >= _S_L1, i < _S_S2))
    def _():
        j = i - _S_L1
        ht = inter_edge(w1t_ref[...], j)
        x2s[pl.ds(j * _CHL, _CHL), :] = jnp.transpose(
            (ht * ht).astype(bf16))

    @pl.when(i == _S_S2)
    def _():
        s = jax.lax.dot_general(inct_bf[...], x2s[...], _DNAT,
                                preferred_element_type=_F32)
        ehi[...] = (s * ide[...]).astype(bf16)

    @pl.when(i == _S_T2)
    def _():
        pooled[...] = jnp.zeros((_D, 1), _F32)

    @pl.when(jnp.logical_and(i >= _S_L2, i < _S_HEAD))
    def _():
        j = i - _S_L2
        ht = inter_edge(w2t_ref[...], j)
        pooled[...] = jnp.maximum(pooled[...],
                                  jnp.max(ht, axis=1, keepdims=True))

    @pl.when(i == _S_HEAD)
    def _():
        out_ref[...] = (
            jax.lax.dot_general(wlin_ref[...], pooled[...], _DNAT,
                                preferred_element_type=_F32)
            + blin_ref[...])


@jax.jit
def kernel(x_0, incidence, W1, W2, Wlin, b_lin):
    def x_map(i):
        return (jnp.where(i < _S_STREAM, i, 0), 0)

    def inct_map(i):
        inside = jnp.logical_and(i >= _S_STREAM, i < _S_T1)
        return (jnp.where(inside, i - _S_STREAM, 0), 0)

    def const_map(i):
        return (0, 0)

    out = pl.pallas_call(
        _hypersage_kernel,
        grid=(_STEPS,),
        in_specs=[
            pl.BlockSpec((_CHN, _D), x_map),
            pl.BlockSpec((_CHE, _N), inct_map),
            pl.BlockSpec((_D, _D), const_map),
            pl.BlockSpec((_D, _D), const_map),
            pl.BlockSpec((_D, _D), const_map),
            pl.BlockSpec((_D, 1), const_map),
        ],
        out_specs=pl.BlockSpec((_D, 1), const_map),
        out_shape=jax.ShapeDtypeStruct((_D, 1), jnp.float32),
        scratch_shapes=[
            pltpu.VMEM((_E, _NP), jnp.bfloat16),
            pltpu.VMEM((_NP, _D), jnp.bfloat16),
            pltpu.VMEM((_E, _D), jnp.bfloat16),
            pltpu.VMEM((_D, _E), jnp.bfloat16),
            pltpu.VMEM((_E, 1), _F32),
            pltpu.VMEM((_D, 1), _F32),
        ],
        compiler_params=pltpu.CompilerParams(
            vmem_limit_bytes=64 * 1024 * 1024,
        ),
    )(x_0, incidence.T, W1.T, W2.T, Wlin, b_lin.reshape(-1, 1))
    return out.reshape(-1)


# speedup vs baseline: 2.3707x; 1.0010x over previous
import jax
import jax.numpy as jnp
from jax.experimental import pallas as pl
from jax.experimental.pallas import tpu as pltpu

_N = 10000
_NP = 10240
_E = 2000
_D = 128
_CHN = 1000
_CHE = 80
_CHL = 1280
_NL = _NP // _CHL
_NX = _N // _CHN
_NE = _E // _CHE
_S_STREAM = _NX
_S_T1 = _S_STREAM + _NE
_S_L1 = _S_T1 + 1
_S_S2 = _S_L1 + _NL
_S_T2 = _S_S2 + 1
_S_L2 = _S_T2 + 1
_S_HEAD = _S_L2 + _NL
_STEPS = _S_HEAD + 1
_DNAT = (((1,), (0,)), ((), ()))
_F32 = jnp.float32


def _hypersage_kernel(x_ref, inct_ref, w1t_ref, w2t_ref, wlin_ref, blin_ref,
                      out_ref, inct_bf, x2s, ehi, ehit, ide, pooled):
    i = pl.program_id(0)
    bf16 = jnp.bfloat16

    @pl.when(i == 0)
    def _():
        x2s[pl.ds(_N, _NP - _N), :] = jnp.zeros((_NP - _N, _D), bf16)

    @pl.when(i < _S_STREAM)
    def _():
        f = x_ref[...]
        x2s[pl.ds(i * _CHN, _CHN), :] = (f * f).astype(bf16)

    @pl.when(jnp.logical_and(i >= _S_STREAM, i < _S_T1))
    def _():
        e = i - _S_STREAM
        re = pl.ds(e * _CHE, _CHE)
        blk = inct_ref[...]
        bbp = jnp.concatenate(
            [blk.astype(bf16), jnp.zeros((_CHE, _NP - _N), bf16)], axis=1)
        inct_bf[re, :] = bbp
        inv_deg = 1.0 / jnp.sum(blk, axis=1, keepdims=True)
        ide[re, :] = inv_deg
        s = jax.lax.dot_general(bbp, x2s[...], _DNAT,
                                preferred_element_type=_F32)
        ehi[re, :] = (s * inv_deg).astype(bf16)

    @pl.when(jnp.logical_or(i == _S_T1, i == _S_T2))
    def _():
        ehit[...] = jnp.transpose(ehi[...])

    def inter_edge(Wt, j):
        rl = pl.ds(j * _CHL, _CHL)
        tt = jax.lax.dot_general(ehit[...], inct_bf[:, rl], _DNAT,
                                 preferred_element_type=_F32)
        ht = jax.lax.dot_general(Wt, jnp.sqrt(tt), _DNAT,
                                 preferred_element_type=_F32)
        ht = jnp.maximum(ht, 0.0)
        norm = jnp.sqrt(jnp.sum(ht * ht, axis=0, keepdims=True))
        return ht / (norm + 1e-12)

    @pl.when(jnp.logical_and(i >= _S_L1, i < _S_S2))
    def _():
        j = i - _S_L1
        ht = inter_edge(w1t_ref[...], j)
        x2s[pl.ds(j * _CHL, _CHL), :] = jnp.transpose(
            (ht * ht).astype(bf16))

    @pl.when(i == _S_S2)
    def _():
        s = jax.lax.dot_general(inct_bf[...], x2s[...], _DNAT,
                                preferred_element_type=_F32)
        ehi[...] = (s * ide[...]).astype(bf16)

    @pl.when(i == _S_T2)
    def _():
        pooled[...] = jnp.zeros((_D, 1), _F32)

    @pl.when(jnp.logical_and(i >= _S_L2, i < _S_HEAD))
    def _():
        j = i - _S_L2
        ht = inter_edge(w2t_ref[...], j)
        pooled[...] = jnp.maximum(pooled[...],
                                  jnp.max(ht, axis=1, keepdims=True))

    @pl.when(i == _S_HEAD)
    def _():
        out_ref[...] = (
            jax.lax.dot_general(wlin_ref[...], pooled[...], _DNAT,
                                preferred_element_type=_F32)
            + blin_ref[...])


@jax.jit
def kernel(x_0, incidence, W1, W2, Wlin, b_lin):
    def x_map(i):
        return (jnp.where(i < _S_STREAM, i, 0), 0)

    def inct_map(i):
        inside = jnp.logical_and(i >= _S_STREAM, i < _S_T1)
        return (jnp.where(inside, i - _S_STREAM, 0), 0)

    def const_map(i):
        return (0, 0)

    out = pl.pallas_call(
        _hypersage_kernel,
        grid=(_STEPS,),
        in_specs=[
            pl.BlockSpec((_CHN, _D), x_map),
            pl.BlockSpec((_CHE, _N), inct_map),
            pl.BlockSpec((_D, _D), const_map),
            pl.BlockSpec((_D, _D), const_map),
            pl.BlockSpec((_D, _D), const_map),
            pl.BlockSpec((_D, 1), const_map),
        ],
        out_specs=pl.BlockSpec((_D, 1), const_map),
        out_shape=jax.ShapeDtypeStruct((_D, 1), jnp.float32),
        scratch_shapes=[
            pltpu.VMEM((_E, _NP), jnp.bfloat16),
            pltpu.VMEM((_NP, _D), jnp.bfloat16),
            pltpu.VMEM((_E, _D), jnp.bfloat16),
            pltpu.VMEM((_D, _E), jnp.bfloat16),
            pltpu.VMEM((_E, 1), _F32),
            pltpu.VMEM((_D, 1), _F32),
        ],
        compiler_params=pltpu.CompilerParams(
            vmem_limit_bytes=64 * 1024 * 1024,
        ),
    )(x_0, incidence.T, W1.T, W2.T, Wlin, b_lin.reshape(-1, 1))
    return out.reshape(-1)
